# baseline (speedup 1.0000x reference)
import functools

import jax
import jax.numpy as jnp
from jax import lax
from jax.experimental import pallas as pl
from jax.experimental.pallas import tpu as pltpu
from jax.experimental.pallas import tpu_sc as plsc

N = 31744
E = 507904
D = 128
H = 64
QH = 32
NT = 16
NC = 2
CH = 128
G = 4
RPT = N // NT
EPT = E // NT
NCH = EPT // CH
NBLK = NCH // G
EPTC = E // (NC * NT)
NCHD = EPTC // CH

@functools.lru_cache(maxsize=None)
def _mesh():
    return plsc.VectorSubcoreMesh(core_axis_name="c", subcore_axis_name="s",
                                  num_cores=NC, num_subcores=NT)


WBR = 248


def _prop_pass(u_q, v_h, src_t, dst_t, acc, sbufs, dbufs, rows, zbuf, wbuf,
               isems, gsems, t, p):
    base = t * RPT

    def zcopy(j, carry):
        pltpu.sync_copy(zbuf, acc.at[pl.ds(base + j * WBR, WBR), :])
        return carry

    lax.fori_loop(0, RPT // WBR, zcopy, 0)
    plsc.subcore_barrier()

    pltpu.sync_copy(src_t.at[pl.ds(0, G)], sbufs[0])
    pltpu.sync_copy(dst_t.at[pl.ds(0, G)], dbufs[0])
    pltpu.async_copy(u_q.at[sbufs[0].at[0]], rows[0], gsems[0])

    def outer(m, carry):
        for b in range(2):
            blk = 2 * m + b
            sb, db = sbufs[b], dbufs[b]
            nsb, ndb = sbufs[1 - b], dbufs[1 - b]

            @pl.when(blk + 1 < NBLK)
            def _():
                pltpu.async_copy(src_t.at[pl.ds((blk + 1) * G, G)], nsb,
                                 isems[1 - b])
                pltpu.async_copy(dst_t.at[pl.ds((blk + 1) * G, G)], ndb,
                                 isems[1 - b])

            for g in range(G):
                rb = g % 2
                pltpu.make_async_copy(u_q.at[sb.at[0]], rows[rb],
                                      gsems[rb]).wait()
                if g < G - 1:
                    pltpu.async_copy(u_q.at[sb.at[g + 1]], rows[1 - rb],
                                     gsems[1 - rb])
                else:
                    @pl.when(blk + 1 < NBLK)
                    def _():
                        pltpu.make_async_copy(src_t.at[pl.ds(0, G)], nsb,
                                              isems[1 - b]).wait()
                        pltpu.make_async_copy(dst_t.at[pl.ds(0, G)], ndb,
                                              isems[1 - b]).wait()
                        pltpu.async_copy(u_q.at[nsb.at[0]], rows[1 - rb],
                                         gsems[1 - rb])
                pltpu.sync_copy(rows[rb], acc.at[db.at[g]], add=True)
        return carry

    lax.fori_loop(0, NBLK // 2, outer, 0)
    plsc.subcore_barrier()

    def wb(j, carry):
        r0 = base + j * WBR
        pltpu.sync_copy(acc.at[pl.ds(r0, WBR), :], wbuf)
        pltpu.sync_copy(wbuf, v_h.at[pl.ds(r0, WBR), pl.ds(p * QH, QH)])
        return carry

    lax.fori_loop(0, RPT // WBR, wb, 0)
    plsc.subcore_barrier()


def _prop_half(u_h, v_h, src_r, dst_r, acc, sbufs, dbufs, rows, zbuf, wbuf,
               isems, gsems, t):
    def zstore(i, carry):
        r = i // (QH // 16)
        k = i - r * (QH // 16)
        zbuf[r, pl.ds(k * 16, 16)] = jnp.zeros((16,), jnp.float32)
        return carry

    lax.fori_loop(0, WBR * (QH // 16), zstore, 0)

    for p in range(2):
        _prop_pass(u_h.at[:, pl.ds(p * QH, QH)], v_h, src_r.at[t],
                   dst_r.at[t], acc, sbufs, dbufs, rows, zbuf, wbuf, isems,
                   gsems, t, p)


def _prop_body(u, src_r, dst_r, v, acc, sbuf0, sbuf1, dbuf0, dbuf1, rows0,
               rows1, zbuf, wbuf, isem0, isem1, gsem0, gsem1):
    c = lax.axis_index("c")
    t = lax.axis_index("s")

    @pl.when(c == 0)
    def _():
        _prop_half(u.at[0], v.at[0], src_r, dst_r, acc, (sbuf0, sbuf1),
                   (dbuf0, dbuf1), (rows0, rows1), zbuf, wbuf,
                   (isem0, isem1), (gsem0, gsem1), t)

    @pl.when(c == 1)
    def _():
        _prop_half(u.at[1], v.at[1], src_r, dst_r, acc, (sbuf0, sbuf1),
                   (dbuf0, dbuf1), (rows0, rows1), zbuf, wbuf,
                   (isem0, isem1), (gsem0, gsem1), t)


@functools.lru_cache(maxsize=None)
def _build_sc_prop():
    return pl.kernel(
        _prop_body,
        out_type=jax.ShapeDtypeStruct((NC, N, H), jnp.float32),
        mesh=_mesh(),
        compiler_params=pltpu.CompilerParams(use_tc_tiling_on_sc=False),
        scratch_types=[
            pltpu.VMEM_SHARED((N, QH), jnp.float32),
            pltpu.VMEM((G, CH), jnp.int32),
            pltpu.VMEM((G, CH), jnp.int32),
            pltpu.VMEM((G, CH), jnp.int32),
            pltpu.VMEM((G, CH), jnp.int32),
            pltpu.VMEM((CH, QH), jnp.float32),
            pltpu.VMEM((CH, QH), jnp.float32),
            pltpu.VMEM((WBR, QH), jnp.float32),
            pltpu.VMEM((WBR, QH), jnp.float32),
            pltpu.SemaphoreType.DMA,
            pltpu.SemaphoreType.DMA,
            pltpu.SemaphoreType.DMA,
            pltpu.SemaphoreType.DMA,
        ],
    )


def _sc_prop(u, src_r, dst_r):
    return _build_sc_prop()(u, src_r, dst_r)


def _degree_body(src_r, deg, acc, src_v, ones_v, zbuf, ssem):
    c = lax.axis_index("c")
    t = lax.axis_index("s")
    base = t * RPT

    def zstore(i, carry):
        zbuf[pl.ds(i * 16, 16)] = jnp.zeros((16,), jnp.float32)
        return carry

    lax.fori_loop(0, RPT // 16, zstore, 0)
    pltpu.sync_copy(zbuf, acc.at[pl.ds(base, RPT)])

    def ostore(i, carry):
        ones_v[pl.ds(i * 16, 16)] = jnp.ones((16,), jnp.float32)
        return carry

    lax.fori_loop(0, CH // 16, ostore, 0)
    pltpu.sync_copy(src_r.at[c, t], src_v)
    plsc.subcore_barrier()

    def group(g, carry):
        for b in range(4):
            pltpu.async_copy(ones_v, acc.at[src_v.at[4 * g + b]], ssem,
                             add=True)
        for b in range(4):
            pltpu.make_async_copy(ones_v, acc.at[src_v.at[0]], ssem).wait()
        return carry

    lax.fori_loop(0, NCHD // 4, group, 0)
    plsc.subcore_barrier()
    pltpu.sync_copy(acc.at[pl.ds(base, RPT)], zbuf)
    pltpu.sync_copy(zbuf, deg.at[pl.ds(c * N + base, RPT)])


@functools.lru_cache(maxsize=None)
def _build_sc_degree():
    return pl.kernel(
        _degree_body,
        out_type=jax.ShapeDtypeStruct((NC * N,), jnp.float32),
        mesh=_mesh(),
        compiler_params=pltpu.CompilerParams(use_tc_tiling_on_sc=False),
        scratch_types=[
            pltpu.VMEM_SHARED((N,), jnp.float32),
            pltpu.VMEM((NCHD, CH), jnp.int32),
            pltpu.VMEM((CH,), jnp.float32),
            pltpu.VMEM((RPT,), jnp.float32),
            pltpu.SemaphoreType.DMA,
        ],
    )


def _sc_degree(src_r):
    return _build_sc_degree()(src_r)


def _scol(deg_blk):
    d = deg_blk[0:1, :] + deg_blk[1:2, :]
    s = jnp.where(d > 0, lax.rsqrt(jnp.maximum(d, 1.0)), 0.0)
    return jnp.transpose(jnp.broadcast_to(s, (D, D)))


def _s2col(deg_blk):
    d = deg_blk[0:1, :] + deg_blk[1:2, :]
    s2 = jnp.where(d > 0, 1.0 / jnp.maximum(d, 1.0), 0.0)
    return jnp.transpose(jnp.broadcast_to(s2, (D, D)))


def _prep_kernel(deg_r, x_r, u0_r):
    sc = _scol(deg_r[...])
    u = sc * x_r[...]
    u0_r[0] = u[:, :H]
    u0_r[1] = u[:, H:]


def _scale_kernel(deg_r, v_r, u_r):
    s2 = _s2col(deg_r[...])[:, :H]
    u_r[0] = s2 * v_r[0]
    u_r[1] = s2 * v_r[1]


def _layer_kernel(deg_r, h_r, v1_r, v2_r, w0_r, w1_r, w2_r, b_r, hn_r, un_r):
    sc = _scol(deg_r[...])
    v1c = jnp.concatenate([v1_r[0], v1_r[1]], axis=1)
    v2c = jnp.concatenate([v2_r[0], v2_r[1]], axis=1)
    z = (jnp.dot(v1c, -w1_r[...], preferred_element_type=jnp.float32)
         + jnp.dot(v2c, 2.0 * w2_r[...], preferred_element_type=jnp.float32))
    out = (jnp.dot(h_r[...], w0_r[...] - w2_r[...],
                   preferred_element_type=jnp.float32)
           + sc * z + b_r[...])
    hn = jnp.maximum(out, 0.0)
    hn_r[...] = hn
    su = sc * hn
    un_r[0] = su[:, :H]
    un_r[1] = su[:, H:]


def _att_kernel(f0_r, f1_r, f2_r, f3_r, f4_r, aw_r, ab_r, out_r):
    cat = jnp.concatenate(
        [f0_r[...], f1_r[...], f2_r[...], f3_r[...], f4_r[...]], axis=1)
    a = jnp.dot(cat, aw_r[...], preferred_element_type=jnp.float32) + ab_r[...]
    a = a - jnp.max(a, axis=1, keepdims=True)
    e = jnp.exp(a)
    attw = e / jnp.sum(e, axis=1, keepdims=True)
    out_r[...] = cat * attw


def _mlp1_kernel(a_r, w_r, b_r, acc_r):
    k = pl.program_id(0)

    @pl.when(k == 0)
    def _():
        acc_r[...] = jnp.broadcast_to(b_r[...], acc_r.shape)

    acc_r[...] += jnp.dot(a_r[...], w_r[...],
                          preferred_element_type=jnp.float32)


def _head_kernel(y_r, g1_r, be1_r, w2_r, b2_r, g2_r, be2_r, w3_r, b3_r,
                 out_r):
    def bn(h, g, be):
        mu = jnp.mean(h, axis=0, keepdims=True)
        var = jnp.mean((h - mu) ** 2, axis=0, keepdims=True)
        return g * (h - mu) / jnp.sqrt(var + 1e-5) + be

    h1 = jnp.maximum(bn(y_r[...], g1_r[...], be1_r[...]), 0.0)
    h2 = jnp.dot(h1, w2_r[...], preferred_element_type=jnp.float32) + b2_r[...]
    h2 = jnp.maximum(bn(h2, g2_r[...], be2_r[...]), 0.0)
    logits = (jnp.dot(h2, w3_r[...], preferred_element_type=jnp.float32)
              + b3_r[...])
    m = jnp.max(logits, axis=1, keepdims=True)
    ex = jnp.exp(logits - m)
    out_r[...] = ex / jnp.sum(ex, axis=1, keepdims=True)


_G = N // D

_deg_spec = pl.BlockSpec((NC, D), lambda i: (0, i))
_row_spec = pl.BlockSpec((D, D), lambda i: (i, 0))
_half_spec = pl.BlockSpec((NC, D, H), lambda i: (0, i, 0))
_full128 = pl.BlockSpec((D, D), lambda i: (0, 0))


def _tc_prep(deg2, x):
    return pl.pallas_call(
        _prep_kernel,
        grid=(_G,),
        in_specs=[_deg_spec, _row_spec],
        out_specs=_half_spec,
        out_shape=jax.ShapeDtypeStruct((NC, N, H), jnp.float32),
    )(deg2, x)


def _tc_scale(deg2, v):
    return pl.pallas_call(
        _scale_kernel,
        grid=(_G,),
        in_specs=[_deg_spec, _half_spec],
        out_specs=_half_spec,
        out_shape=jax.ShapeDtypeStruct((NC, N, H), jnp.float32),
    )(deg2, v)


def _tc_layer(deg2, h, v1, v2, w0, w1, w2, b):
    return pl.pallas_call(
        _layer_kernel,
        grid=(_G,),
        in_specs=[_deg_spec, _row_spec, _half_spec, _half_spec,
                  _full128, _full128, _full128,
                  pl.BlockSpec((1, D), lambda i: (0, 0))],
        out_specs=[_row_spec, _half_spec],
        out_shape=[jax.ShapeDtypeStruct((N, D), jnp.float32),
                   jax.ShapeDtypeStruct((NC, N, H), jnp.float32)],
    )(deg2, h, v1, v2, w0, w1, w2, b)


def _tc_att(feats, att_w, att_b):
    att = 5 * D
    return pl.pallas_call(
        _att_kernel,
        grid=(_G,),
        in_specs=[_row_spec] * 5 + [
            pl.BlockSpec((att, att), lambda i: (0, 0)),
            pl.BlockSpec((1, att), lambda i: (0, 0))],
        out_specs=pl.BlockSpec((D, att), lambda i: (i, 0)),
        out_shape=jax.ShapeDtypeStruct((N, att), jnp.float32),
    )(*feats, att_w, att_b)


def _tc_mlp1(flat, w1, b1):
    batch, kdim = flat.shape
    lin = w1.shape[1]
    kb = 3968
    steps = kdim // kb
    return pl.pallas_call(
        _mlp1_kernel,
        grid=(steps,),
        in_specs=[pl.BlockSpec((batch, kb), lambda k: (0, k)),
                  pl.BlockSpec((kb, lin), lambda k: (k, 0)),
                  pl.BlockSpec((1, lin), lambda k: (0, 0))],
        out_specs=pl.BlockSpec((batch, lin), lambda k: (0, 0)),
        out_shape=jax.ShapeDtypeStruct((batch, lin), jnp.float32),
    )(flat, w1, b1)


def _tc_head(y1, g1, be1, w2, b2, g2, be2, w3, b3):
    batch, lin = y1.shape
    lin2 = w2.shape[1]
    hc = w3.shape[1]
    row = lambda a: a.reshape(1, -1)
    return pl.pallas_call(
        _head_kernel,
        grid=(1,),
        in_specs=[pl.BlockSpec((batch, lin), lambda k: (0, 0)),
                  pl.BlockSpec((1, lin), lambda k: (0, 0)),
                  pl.BlockSpec((1, lin), lambda k: (0, 0)),
                  pl.BlockSpec((lin, lin2), lambda k: (0, 0)),
                  pl.BlockSpec((1, lin2), lambda k: (0, 0)),
                  pl.BlockSpec((1, lin2), lambda k: (0, 0)),
                  pl.BlockSpec((1, lin2), lambda k: (0, 0)),
                  pl.BlockSpec((lin2, hc), lambda k: (0, 0)),
                  pl.BlockSpec((1, hc), lambda k: (0, 0))],
        out_specs=pl.BlockSpec((batch, hc), lambda k: (0, 0)),
        out_shape=jax.ShapeDtypeStruct((batch, hc), jnp.float32),
    )(y1, row(g1), row(be1), w2, row(b2), row(g2), row(be2), w3, row(b3))


def _xla_prop(u, src, dst):
    return jnp.stack([
        jax.ops.segment_sum(u[c][src], dst, num_segments=N)
        for c in range(NC)
    ])


def kernel(x, edge_index, conv_w, conv_b, att_w, att_b, w1, b1, g1, be1, w2,
           b2, g2, be2, w3, b3):
    src = edge_index[0]
    dst = edge_index[1]

    deg = jax.ops.segment_sum(jnp.ones((E,), x.dtype), src, num_segments=N)
    deg2 = jnp.stack([deg, jnp.zeros_like(deg)])
    u = _tc_prep(deg2, x)

    h = x
    feats = []
    for i in range(5):
        v1 = _xla_prop(u, src, dst)
        u1 = _tc_scale(deg2, v1)
        v2 = _xla_prop(u1, src, dst)
        h, u = _tc_layer(deg2, h, v1, v2, conv_w[i, 0], conv_w[i, 1],
                         conv_w[i, 2], conv_b[i].reshape(1, D))
        feats.append(h)

    att = _tc_att(feats, att_w, att_b.reshape(1, -1))
    flat = att.reshape(512, -1)
    y1 = _tc_mlp1(flat, w1, b1.reshape(1, -1))
    return _tc_head(y1, g1, be1, w2, b2, g2, be2, w3, b3)


# speedup vs baseline: 1.4878x; 1.4878x over previous
import functools

import jax
import jax.numpy as jnp
from jax import lax
from jax.experimental import pallas as pl
from jax.experimental.pallas import tpu as pltpu
from jax.experimental.pallas import tpu_sc as plsc

N = 31744
E = 507904
D = 128
NC = 2
NT = 16
QN = N // 4
TR = 128
ACCR = QN + TR
CH = 16
BC = 16
BE = BC * CH
EPT = E // NT
NBL = EPT // BE
RPT = QN // NT


@functools.lru_cache(maxsize=None)
def _mesh():
    return plsc.VectorSubcoreMesh(core_axis_name="c", subcore_axis_name="s",
                                  num_cores=NC, num_subcores=NT)


def _prop_core(u, v, didx_q, sidx_t, acc, sbufs, dbufs, rows, isems,
               gsems, q, t):
    base = t * RPT
    didx_t = didx_q.at[t]

    def zstore(i, carry):
        r = i // (D // 16)
        k = i - r * (D // 16)
        rows[1][r, pl.ds(k * 16, 16)] = jnp.zeros((16,), jnp.float32)
        return carry

    lax.fori_loop(0, CH * (D // 16), zstore, 0)

    def zcopy(j, carry):
        pltpu.sync_copy(rows[1], acc.at[pl.ds(base + j * CH, CH), :])
        return carry

    lax.fori_loop(0, RPT // CH, zcopy, 0)
    plsc.subcore_barrier()

    pltpu.sync_copy(sidx_t.at[pl.ds(0, BE)], sbufs[0])
    pltpu.sync_copy(didx_t.at[0], dbufs[0])
    pltpu.async_copy(u.at[sbufs[0].at[pl.ds(0, CH)]], rows[0], gsems[0])

    def outer(m, carry):
        for b in range(2):
            blk = 2 * m + b
            sb, db = sbufs[b], dbufs[b]
            nsb, ndb = sbufs[1 - b], dbufs[1 - b]

            @pl.when(blk + 1 < NBL)
            def _():
                pltpu.async_copy(sidx_t.at[pl.ds((blk + 1) * BE, BE)], nsb,
                                 isems[1 - b])
                pltpu.async_copy(didx_t.at[blk + 1], ndb, isems[1 - b])

            for g in range(BC):
                rb = g % 2
                pltpu.make_async_copy(u.at[sb.at[pl.ds(0, CH)]], rows[rb],
                                      gsems[rb]).wait()
                if g < BC - 1:
                    pltpu.async_copy(u.at[sb.at[pl.ds((g + 1) * CH, CH)]],
                                     rows[1 - rb], gsems[1 - rb])
                else:
                    @pl.when(blk + 1 < NBL)
                    def _():
                        pltpu.make_async_copy(sidx_t.at[pl.ds(0, BE)], nsb,
                                              isems[1 - b]).wait()
                        pltpu.make_async_copy(didx_t.at[0], ndb,
                                              isems[1 - b]).wait()
                        pltpu.async_copy(u.at[nsb.at[pl.ds(0, CH)]],
                                         rows[1 - rb], gsems[1 - rb])
                pltpu.sync_copy(rows[rb], acc.at[db.at[g]], add=True)
        return carry

    lax.fori_loop(0, NBL // 2, outer, 0)
    plsc.subcore_barrier()

    pltpu.sync_copy(acc.at[pl.ds(base, RPT), :],
                    v.at[pl.ds(q * QN + base, RPT), :])
    plsc.subcore_barrier()


def _prop_body(u, sidx, didx, v, acc, sbuf0, sbuf1, dbuf0, dbuf1, rows0,
               rows1, isem0, isem1, gsem0, gsem1):
    c = lax.axis_index("c")
    t = lax.axis_index("s")

    for p in range(2):
        for cc in range(NC):
            @pl.when(c == cc)
            def _():
                _prop_core(u, v, didx.at[2 * cc + p], sidx.at[t], acc,
                           (sbuf0, sbuf1), (dbuf0, dbuf1), (rows0, rows1),
                           (isem0, isem1), (gsem0, gsem1), 2 * cc + p, t)


@functools.lru_cache(maxsize=None)
def _build_sc_prop():
    return pl.kernel(
        _prop_body,
        out_type=jax.ShapeDtypeStruct((N, D), jnp.float32),
        mesh=_mesh(),
        scratch_types=[
            pltpu.VMEM_SHARED((ACCR, D), jnp.float32),
            pltpu.VMEM((BE,), jnp.int32),
            pltpu.VMEM((BE,), jnp.int32),
            pltpu.VMEM((BC, CH), jnp.int32),
            pltpu.VMEM((BC, CH), jnp.int32),
            pltpu.VMEM((CH, D), jnp.float32),
            pltpu.VMEM((CH, D), jnp.float32),
            pltpu.SemaphoreType.DMA,
            pltpu.SemaphoreType.DMA,
            pltpu.SemaphoreType.DMA,
            pltpu.SemaphoreType.DMA,
        ],
    )


def _sc_prop(u, sidx, didx):
    return _build_sc_prop()(u, sidx, didx)


def _scol(deg_blk):
    return jnp.where(deg_blk > 0, lax.rsqrt(jnp.maximum(deg_blk, 1.0)), 0.0)


def _s2col(deg_blk):
    return jnp.where(deg_blk > 0, 1.0 / jnp.maximum(deg_blk, 1.0), 0.0)


def _didx_kernel(d_r, q0_r, q1_r, q2_r, q3_r):
    d = d_r[...]
    trash = QN + lax.broadcasted_iota(jnp.int32, d.shape, 1) % TR
    for q, out_r in enumerate((q0_r, q1_r, q2_r, q3_r)):
        dq = d - q * QN
        out_r[...] = jnp.where((dq >= 0) & (dq < QN), dq, trash)


def _prep_kernel(deg_r, x_r, u0_r):
    u0_r[...] = _scol(deg_r[...]) * x_r[...]


def _scale_kernel(deg_r, v_r, u_r):
    u_r[...] = _s2col(deg_r[...]) * v_r[...]


def _layer_kernel(deg_r, h_r, v1_r, v2_r, w0_r, w1_r, w2_r, b_r, hn_r, un_r):
    sc = _scol(deg_r[...])
    z = (jnp.dot(v1_r[...], -w1_r[...], preferred_element_type=jnp.float32)
         + jnp.dot(v2_r[...], 2.0 * w2_r[...],
                   preferred_element_type=jnp.float32))
    out = (jnp.dot(h_r[...], w0_r[...] - w2_r[...],
                   preferred_element_type=jnp.float32)
           + sc * z + b_r[...])
    hn = jnp.maximum(out, 0.0)
    hn_r[...] = hn
    un_r[...] = sc * hn


def _att_kernel(f0_r, f1_r, f2_r, f3_r, f4_r, aw_r, ab_r, out_r):
    cat = jnp.concatenate(
        [f0_r[...], f1_r[...], f2_r[...], f3_r[...], f4_r[...]], axis=1)
    a = jnp.dot(cat, aw_r[...], preferred_element_type=jnp.float32) + ab_r[...]
    a = a - jnp.max(a, axis=1, keepdims=True)
    e = jnp.exp(a)
    attw = e / jnp.sum(e, axis=1, keepdims=True)
    out_r[...] = cat * attw


def _mlp1_kernel(a_r, w_r, b_r, acc_r):
    k = pl.program_id(0)

    @pl.when(k == 0)
    def _():
        acc_r[...] = jnp.broadcast_to(b_r[...], acc_r.shape)

    acc_r[...] += jnp.dot(a_r[...], w_r[...],
                          preferred_element_type=jnp.float32)


def _head_kernel(y_r, g1_r, be1_r, w2_r, b2_r, g2_r, be2_r, w3_r, b3_r,
                 out_r):
    def bn(h, g, be):
        mu = jnp.mean(h, axis=0, keepdims=True)
        var = jnp.mean((h - mu) ** 2, axis=0, keepdims=True)
        return g * (h - mu) / jnp.sqrt(var + 1e-5) + be

    h1 = jnp.maximum(bn(y_r[...], g1_r[...], be1_r[...]), 0.0)
    h2 = jnp.dot(h1, w2_r[...], preferred_element_type=jnp.float32) + b2_r[...]
    h2 = jnp.maximum(bn(h2, g2_r[...], be2_r[...]), 0.0)
    logits = (jnp.dot(h2, w3_r[...], preferred_element_type=jnp.float32)
              + b3_r[...])
    m = jnp.max(logits, axis=1, keepdims=True)
    ex = jnp.exp(logits - m)
    out_r[...] = ex / jnp.sum(ex, axis=1, keepdims=True)


_G = N // D

_row_spec = pl.BlockSpec((D, D), lambda i: (i, 0))
_full128 = pl.BlockSpec((D, D), lambda i: (0, 0))

_EB = E // D


def _tc_didx(idx):
    qs = pl.pallas_call(
        _didx_kernel,
        grid=(_EB // D,),
        in_specs=[_row_spec],
        out_specs=[_row_spec] * 4,
        out_shape=[jax.ShapeDtypeStruct((_EB, D), jnp.int32)] * 4,
    )(idx.reshape(_EB, D))
    shape = (NT, NBL, BC, CH)
    return jnp.stack([a.reshape(shape) for a in qs])


def _tc_prep(deg, x):
    return pl.pallas_call(
        _prep_kernel,
        grid=(_G,),
        in_specs=[_row_spec, _row_spec],
        out_specs=_row_spec,
        out_shape=jax.ShapeDtypeStruct((N, D), jnp.float32),
    )(deg, x)


def _tc_scale(deg, v):
    return pl.pallas_call(
        _scale_kernel,
        grid=(_G,),
        in_specs=[_row_spec, _row_spec],
        out_specs=_row_spec,
        out_shape=jax.ShapeDtypeStruct((N, D), jnp.float32),
    )(deg, v)


def _tc_layer(deg, h, v1, v2, w0, w1, w2, b):
    return pl.pallas_call(
        _layer_kernel,
        grid=(_G,),
        in_specs=[_row_spec, _row_spec, _row_spec, _row_spec,
                  _full128, _full128, _full128,
                  pl.BlockSpec((1, D), lambda i: (0, 0))],
        out_specs=[_row_spec, _row_spec],
        out_shape=[jax.ShapeDtypeStruct((N, D), jnp.float32),
                   jax.ShapeDtypeStruct((N, D), jnp.float32)],
    )(deg, h, v1, v2, w0, w1, w2, b)


def _tc_att(feats, att_w, att_b):
    att = 5 * D
    return pl.pallas_call(
        _att_kernel,
        grid=(_G,),
        in_specs=[_row_spec] * 5 + [
            pl.BlockSpec((att, att), lambda i: (0, 0)),
            pl.BlockSpec((1, att), lambda i: (0, 0))],
        out_specs=pl.BlockSpec((D, att), lambda i: (i, 0)),
        out_shape=jax.ShapeDtypeStruct((N, att), jnp.float32),
    )(*feats, att_w, att_b)


def _tc_mlp1(flat, w1, b1):
    batch, kdim = flat.shape
    lin = w1.shape[1]
    kb = 3968
    steps = kdim // kb
    return pl.pallas_call(
        _mlp1_kernel,
        grid=(steps,),
        in_specs=[pl.BlockSpec((batch, kb), lambda k: (0, k)),
                  pl.BlockSpec((kb, lin), lambda k: (k, 0)),
                  pl.BlockSpec((1, lin), lambda k: (0, 0))],
        out_specs=pl.BlockSpec((batch, lin), lambda k: (0, 0)),
        out_shape=jax.ShapeDtypeStruct((batch, lin), jnp.float32),
    )(flat, w1, b1)


def _tc_head(y1, g1, be1, w2, b2, g2, be2, w3, b3):
    batch, lin = y1.shape
    lin2 = w2.shape[1]
    hc = w3.shape[1]
    row = lambda a: a.reshape(1, -1)
    return pl.pallas_call(
        _head_kernel,
        grid=(1,),
        in_specs=[pl.BlockSpec((batch, lin), lambda k: (0, 0)),
                  pl.BlockSpec((1, lin), lambda k: (0, 0)),
                  pl.BlockSpec((1, lin), lambda k: (0, 0)),
                  pl.BlockSpec((lin, lin2), lambda k: (0, 0)),
                  pl.BlockSpec((1, lin2), lambda k: (0, 0)),
                  pl.BlockSpec((1, lin2), lambda k: (0, 0)),
                  pl.BlockSpec((1, lin2), lambda k: (0, 0)),
                  pl.BlockSpec((lin2, hc), lambda k: (0, 0)),
                  pl.BlockSpec((1, hc), lambda k: (0, 0))],
        out_specs=pl.BlockSpec((batch, hc), lambda k: (0, 0)),
        out_shape=jax.ShapeDtypeStruct((batch, hc), jnp.float32),
    )(y1, row(g1), row(be1), w2, row(b2), row(g2), row(be2), w3, row(b3))


def kernel(x, edge_index, conv_w, conv_b, att_w, att_b, w1, b1, g1, be1, w2,
           b2, g2, be2, w3, b3):
    src = edge_index[0]
    dst = edge_index[1]
    sidx = src.reshape(NT, EPT)
    didx = _tc_didx(dst)
    srct = _tc_didx(src)

    ones_nd = jnp.ones((N, D), jnp.float32)
    deg = _sc_prop(ones_nd, sidx, srct)
    u = _tc_prep(deg, x)

    h = x
    feats = []
    for i in range(5):
        v1 = _sc_prop(u, sidx, didx)
        u1 = _tc_scale(deg, v1)
        v2 = _sc_prop(u1, sidx, didx)
        h, u = _tc_layer(deg, h, v1, v2, conv_w[i, 0], conv_w[i, 1],
                         conv_w[i, 2], conv_b[i].reshape(1, D))
        feats.append(h)

    att = _tc_att(feats, att_w, att_b.reshape(1, -1))
    flat = att.reshape(512, -1)
    y1 = _tc_mlp1(flat, w1, b1.reshape(1, -1))
    return _tc_head(y1, g1, be1, w2, b2, g2, be2, w3, b3)


# speedup vs baseline: 1.6733x; 1.1247x over previous
import functools

import jax
import jax.numpy as jnp
from jax import lax
from jax.experimental import pallas as pl
from jax.experimental.pallas import tpu as pltpu
from jax.experimental.pallas import tpu_sc as plsc

N = 31744
E = 507904
D = 128
NC = 2
NT = 16
HN = N // 2
TR = 80
ACCR = HN + TR
CH = 8
BC = 4
BE = BC * CH
EPT = E // NT
NBL = EPT // BE
RPT = HN // NT


@functools.lru_cache(maxsize=None)
def _mesh():
    return plsc.VectorSubcoreMesh(core_axis_name="c", subcore_axis_name="s",
                                  num_cores=NC, num_subcores=NT)


def _prop_core(u, v, didx_q, sidx_t, acc, sbufs, dbufs, rows, isems,
               gsems, q, t):
    base = t * RPT
    didx_t = didx_q.at[t]

    def zstore(i, carry):
        r = i // (D // 16)
        k = i - r * (D // 16)
        rows[1][r, pl.ds(k * 16, 16)] = jnp.zeros((16,), jnp.float32)
        return carry

    lax.fori_loop(0, CH * (D // 16), zstore, 0)

    def zcopy(j, carry):
        pltpu.sync_copy(rows[1], acc.at[pl.ds(base + j * CH, CH), :])
        return carry

    lax.fori_loop(0, RPT // CH, zcopy, 0)
    plsc.subcore_barrier()

    pltpu.sync_copy(sidx_t.at[pl.ds(0, BE)], sbufs[0])
    pltpu.sync_copy(didx_t.at[0], dbufs[0])
    pltpu.async_copy(u.at[sbufs[0].at[pl.ds(0, CH)]], rows[0], gsems[0])

    def outer(m, carry):
        for b in range(2):
            blk = 2 * m + b
            sb, db = sbufs[b], dbufs[b]
            nsb, ndb = sbufs[1 - b], dbufs[1 - b]

            @pl.when(blk + 1 < NBL)
            def _():
                pltpu.async_copy(sidx_t.at[pl.ds((blk + 1) * BE, BE)], nsb,
                                 isems[1 - b])
                pltpu.async_copy(didx_t.at[blk + 1], ndb, isems[1 - b])

            for g in range(BC):
                rb = g % 2
                pltpu.make_async_copy(u.at[sb.at[pl.ds(0, CH)]], rows[rb],
                                      gsems[rb]).wait()
                if g < BC - 1:
                    pltpu.async_copy(u.at[sb.at[pl.ds((g + 1) * CH, CH)]],
                                     rows[1 - rb], gsems[1 - rb])
                else:
                    @pl.when(blk + 1 < NBL)
                    def _():
                        pltpu.make_async_copy(sidx_t.at[pl.ds(0, BE)], nsb,
                                              isems[1 - b]).wait()
                        pltpu.make_async_copy(didx_t.at[0], ndb,
                                              isems[1 - b]).wait()
                        pltpu.async_copy(u.at[nsb.at[pl.ds(0, CH)]],
                                         rows[1 - rb], gsems[1 - rb])
                pltpu.sync_copy(rows[rb], acc.at[db.at[g]], add=True)
        return carry

    lax.fori_loop(0, NBL // 2, outer, 0)
    plsc.subcore_barrier()

    pltpu.sync_copy(acc.at[pl.ds(base, RPT), :],
                    v.at[pl.ds(q * HN + base, RPT), :])
    plsc.subcore_barrier()


def _prop_body(u, sidx, didx, v, acc, sbuf0, sbuf1, dbuf0, dbuf1, rows0,
               rows1, isem0, isem1, gsem0, gsem1):
    c = lax.axis_index("c")
    t = lax.axis_index("s")

    for cc in range(NC):
        @pl.when(c == cc)
        def _():
            _prop_core(u, v, didx.at[cc], sidx.at[t], acc,
                       (sbuf0, sbuf1), (dbuf0, dbuf1), (rows0, rows1),
                       (isem0, isem1), (gsem0, gsem1), cc, t)


@functools.lru_cache(maxsize=None)
def _build_sc_prop():
    return pl.kernel(
        _prop_body,
        out_type=jax.ShapeDtypeStruct((N, D), jnp.float32),
        mesh=_mesh(),
        scratch_types=[
            pltpu.VMEM_SHARED((ACCR, D), jnp.float32),
            pltpu.VMEM((BE,), jnp.int32),
            pltpu.VMEM((BE,), jnp.int32),
            pltpu.VMEM((BC, CH), jnp.int32),
            pltpu.VMEM((BC, CH), jnp.int32),
            pltpu.VMEM((CH, D), jnp.float32),
            pltpu.VMEM((CH, D), jnp.float32),
            pltpu.SemaphoreType.DMA,
            pltpu.SemaphoreType.DMA,
            pltpu.SemaphoreType.DMA,
            pltpu.SemaphoreType.DMA,
        ],
    )


def _sc_prop(u, sidx, didx):
    return _build_sc_prop()(u, sidx, didx)


def _scol(deg_blk):
    return jnp.where(deg_blk > 0, lax.rsqrt(jnp.maximum(deg_blk, 1.0)), 0.0)


def _s2col(deg_blk):
    return jnp.where(deg_blk > 0, 1.0 / jnp.maximum(deg_blk, 1.0), 0.0)


def _didx_kernel(d_r, q0_r, q1_r):
    d = d_r[...]
    trash = HN + lax.broadcasted_iota(jnp.int32, d.shape, 1) % TR
    for q, out_r in enumerate((q0_r, q1_r)):
        dq = d - q * HN
        out_r[...] = jnp.where((dq >= 0) & (dq < HN), dq, trash)


def _prep_kernel(deg_r, x_r, u0_r):
    u0_r[...] = _scol(deg_r[...]) * x_r[...]


def _scale_kernel(deg_r, v_r, u_r):
    u_r[...] = _s2col(deg_r[...]) * v_r[...]


def _layer_kernel(deg_r, h_r, v1_r, v2_r, w0_r, w1_r, w2_r, b_r, hn_r, un_r):
    sc = _scol(deg_r[...])
    z = (jnp.dot(v1_r[...], -w1_r[...], preferred_element_type=jnp.float32)
         + jnp.dot(v2_r[...], 2.0 * w2_r[...],
                   preferred_element_type=jnp.float32))
    out = (jnp.dot(h_r[...], w0_r[...] - w2_r[...],
                   preferred_element_type=jnp.float32)
           + sc * z + b_r[...])
    hn = jnp.maximum(out, 0.0)
    hn_r[...] = hn
    un_r[...] = sc * hn


def _att_kernel(f0_r, f1_r, f2_r, f3_r, f4_r, aw_r, ab_r, out_r):
    cat = jnp.concatenate(
        [f0_r[...], f1_r[...], f2_r[...], f3_r[...], f4_r[...]], axis=1)
    a = jnp.dot(cat, aw_r[...], preferred_element_type=jnp.float32) + ab_r[...]
    a = a - jnp.max(a, axis=1, keepdims=True)
    e = jnp.exp(a)
    attw = e / jnp.sum(e, axis=1, keepdims=True)
    out_r[...] = cat * attw


def _mlp1_kernel(a_r, w_r, b_r, acc_r):
    k = pl.program_id(0)

    @pl.when(k == 0)
    def _():
        acc_r[...] = jnp.broadcast_to(b_r[...], acc_r.shape)

    acc_r[...] += jnp.dot(a_r[...], w_r[...],
                          preferred_element_type=jnp.float32)


def _head_kernel(y_r, g1_r, be1_r, w2_r, b2_r, g2_r, be2_r, w3_r, b3_r,
                 out_r):
    def bn(h, g, be):
        mu = jnp.mean(h, axis=0, keepdims=True)
        var = jnp.mean((h - mu) ** 2, axis=0, keepdims=True)
        return g * (h - mu) / jnp.sqrt(var + 1e-5) + be

    h1 = jnp.maximum(bn(y_r[...], g1_r[...], be1_r[...]), 0.0)
    h2 = jnp.dot(h1, w2_r[...], preferred_element_type=jnp.float32) + b2_r[...]
    h2 = jnp.maximum(bn(h2, g2_r[...], be2_r[...]), 0.0)
    logits = (jnp.dot(h2, w3_r[...], preferred_element_type=jnp.float32)
              + b3_r[...])
    m = jnp.max(logits, axis=1, keepdims=True)
    ex = jnp.exp(logits - m)
    out_r[...] = ex / jnp.sum(ex, axis=1, keepdims=True)


_G = N // D

_row_spec = pl.BlockSpec((D, D), lambda i: (i, 0))
_full128 = pl.BlockSpec((D, D), lambda i: (0, 0))

_EB = E // D


def _tc_didx(idx):
    qs = pl.pallas_call(
        _didx_kernel,
        grid=(_EB // D,),
        in_specs=[_row_spec],
        out_specs=[_row_spec] * 2,
        out_shape=[jax.ShapeDtypeStruct((_EB, D), jnp.int32)] * 2,
    )(idx.reshape(_EB, D))
    shape = (NT, NBL, BC, CH)
    return jnp.stack([a.reshape(shape) for a in qs])


def _tc_prep(deg, x):
    return pl.pallas_call(
        _prep_kernel,
        grid=(_G,),
        in_specs=[_row_spec, _row_spec],
        out_specs=_row_spec,
        out_shape=jax.ShapeDtypeStruct((N, D), jnp.float32),
    )(deg, x)


def _tc_scale(deg, v):
    return pl.pallas_call(
        _scale_kernel,
        grid=(_G,),
        in_specs=[_row_spec, _row_spec],
        out_specs=_row_spec,
        out_shape=jax.ShapeDtypeStruct((N, D), jnp.float32),
    )(deg, v)


def _tc_layer(deg, h, v1, v2, w0, w1, w2, b):
    return pl.pallas_call(
        _layer_kernel,
        grid=(_G,),
        in_specs=[_row_spec, _row_spec, _row_spec, _row_spec,
                  _full128, _full128, _full128,
                  pl.BlockSpec((1, D), lambda i: (0, 0))],
        out_specs=[_row_spec, _row_spec],
        out_shape=[jax.ShapeDtypeStruct((N, D), jnp.float32),
                   jax.ShapeDtypeStruct((N, D), jnp.float32)],
    )(deg, h, v1, v2, w0, w1, w2, b)


def _tc_att(feats, att_w, att_b):
    att = 5 * D
    return pl.pallas_call(
        _att_kernel,
        grid=(_G,),
        in_specs=[_row_spec] * 5 + [
            pl.BlockSpec((att, att), lambda i: (0, 0)),
            pl.BlockSpec((1, att), lambda i: (0, 0))],
        out_specs=pl.BlockSpec((D, att), lambda i: (i, 0)),
        out_shape=jax.ShapeDtypeStruct((N, att), jnp.float32),
    )(*feats, att_w, att_b)


def _tc_mlp1(flat, w1, b1):
    batch, kdim = flat.shape
    lin = w1.shape[1]
    kb = 3968
    steps = kdim // kb
    return pl.pallas_call(
        _mlp1_kernel,
        grid=(steps,),
        in_specs=[pl.BlockSpec((batch, kb), lambda k: (0, k)),
                  pl.BlockSpec((kb, lin), lambda k: (k, 0)),
                  pl.BlockSpec((1, lin), lambda k: (0, 0))],
        out_specs=pl.BlockSpec((batch, lin), lambda k: (0, 0)),
        out_shape=jax.ShapeDtypeStruct((batch, lin), jnp.float32),
    )(flat, w1, b1)


def _tc_head(y1, g1, be1, w2, b2, g2, be2, w3, b3):
    batch, lin = y1.shape
    lin2 = w2.shape[1]
    hc = w3.shape[1]
    row = lambda a: a.reshape(1, -1)
    return pl.pallas_call(
        _head_kernel,
        grid=(1,),
        in_specs=[pl.BlockSpec((batch, lin), lambda k: (0, 0)),
                  pl.BlockSpec((1, lin), lambda k: (0, 0)),
                  pl.BlockSpec((1, lin), lambda k: (0, 0)),
                  pl.BlockSpec((lin, lin2), lambda k: (0, 0)),
                  pl.BlockSpec((1, lin2), lambda k: (0, 0)),
                  pl.BlockSpec((1, lin2), lambda k: (0, 0)),
                  pl.BlockSpec((1, lin2), lambda k: (0, 0)),
                  pl.BlockSpec((lin2, hc), lambda k: (0, 0)),
                  pl.BlockSpec((1, hc), lambda k: (0, 0))],
        out_specs=pl.BlockSpec((batch, hc), lambda k: (0, 0)),
        out_shape=jax.ShapeDtypeStruct((batch, hc), jnp.float32),
    )(y1, row(g1), row(be1), w2, row(b2), row(g2), row(be2), w3, row(b3))


def kernel(x, edge_index, conv_w, conv_b, att_w, att_b, w1, b1, g1, be1, w2,
           b2, g2, be2, w3, b3):
    src = edge_index[0]
    dst = edge_index[1]
    sidx = src.reshape(NT, EPT)
    didx = _tc_didx(dst)
    srct = _tc_didx(src)

    ones_nd = jnp.ones((N, D), jnp.float32)
    deg = _sc_prop(ones_nd, sidx, srct)
    u = _tc_prep(deg, x)

    h = x
    feats = []
    for i in range(5):
        v1 = _sc_prop(u, sidx, didx)
        u1 = _tc_scale(deg, v1)
        v2 = _sc_prop(u1, sidx, didx)
        h, u = _tc_layer(deg, h, v1, v2, conv_w[i, 0], conv_w[i, 1],
                         conv_w[i, 2], conv_b[i].reshape(1, D))
        feats.append(h)

    att = _tc_att(feats, att_w, att_b.reshape(1, -1))
    flat = att.reshape(512, -1)
    y1 = _tc_mlp1(flat, w1, b1.reshape(1, -1))
    return _tc_head(y1, g1, be1, w2, b2, g2, be2, w3, b3)
